# Initial kernel scaffold; baseline (speedup 1.0000x reference)
import jax
import jax.numpy as jnp
from jax.experimental import pallas as pl


def kernel(x, edge_index, edge_weight, W1, b1, gn_weight, gn_bias, gn_mean_scale, W2, b2):
    raise NotImplementedError("write your pallas kernel here")


# speedup vs baseline: 9.0952x; 9.0952x over previous
import functools

import jax
import jax.numpy as jnp
from jax import lax
from jax.experimental import pallas as pl
from jax.experimental.pallas import tpu as pltpu
from jax.experimental.pallas import tpu_sc as plsc

N = 10000
E = 320000
D = 128

NC = 2
NS = 16
NW = NC * NS
LANES = 16
EPT = E // NW
C = 80
NCHUNK = EPT // C
RPT = N // NS
FR = 125

_MESH = plsc.VectorSubcoreMesh(core_axis_name="c", subcore_axis_name="s")


def _deg_body(dst_hbm, ew_hbm, out_hbm, deg_v, dst_v, ew_v):
    cid = lax.axis_index("c")
    sid = lax.axis_index("s")
    wid = sid * NC + cid
    zero16 = jnp.zeros((LANES,), jnp.float32)

    def _zero(i, carry):
        deg_v[pl.ds(i * LANES, LANES)] = zero16
        return carry

    lax.fori_loop(0, N // LANES, _zero, 0)

    base = wid * EPT

    def _chunk(i, carry):
        pltpu.sync_copy(dst_hbm.at[pl.ds(base + i * C, C)], dst_v)
        pltpu.sync_copy(ew_hbm.at[pl.ds(base + i * C, C)], ew_v)

        def _grp(j, c2):
            idx = dst_v[pl.ds(j * LANES, LANES)]
            w = ew_v[pl.ds(j * LANES, LANES)]
            plsc.addupdate_scatter(deg_v, [idx], w)
            return c2

        lax.fori_loop(0, C // LANES, _grp, 0)
        return carry

    lax.fori_loop(0, NCHUNK, _chunk, 0)
    pltpu.sync_copy(deg_v, out_hbm.at[wid])


_SC_PARAMS = pltpu.CompilerParams(needs_layout_passes=False,
                                  use_tc_tiling_on_sc=False)

_deg_call = pl.kernel(
    _deg_body,
    out_type=jax.ShapeDtypeStruct((NW, N), jnp.float32),
    mesh=_MESH,
    compiler_params=_SC_PARAMS,
    scratch_types=[
        pltpu.VMEM((N,), jnp.float32),
        pltpu.VMEM((C,), jnp.int32),
        pltpu.VMEM((C,), jnp.float32),
    ],
)


def _agg_body(g_hbm, src_hbm, dst_hbm, ew_hbm, out_hbm,
              acc, src_v, dst_v, ew_v, rows_v, zbuf, sem):
    cid = lax.axis_index("c")
    sid = lax.axis_index("s")
    wid = sid * NC + cid
    zero16 = jnp.zeros((LANES,), jnp.float32)

    def _zr(r, carry):
        def _zc(cb, c2):
            zbuf[r, pl.ds(cb * LANES, LANES)] = zero16
            return c2
        lax.fori_loop(0, D // LANES, _zc, 0)
        return carry

    lax.fori_loop(0, FR, _zr, 0)

    def _za(k, carry):
        pltpu.sync_copy(zbuf, acc.at[pl.ds(sid * RPT + k * FR, FR)])
        return carry

    lax.fori_loop(0, RPT // FR, _za, 0)
    plsc.subcore_barrier()

    base = wid * EPT

    def _chunk(i, carry):
        off = base + i * C
        pltpu.sync_copy(src_hbm.at[pl.ds(off, C)], src_v)
        pltpu.sync_copy(dst_hbm.at[pl.ds(off, C)], dst_v)
        pltpu.sync_copy(ew_hbm.at[pl.ds(off, C)], ew_v)
        pltpu.async_copy(g_hbm.at[src_v], rows_v, sem).wait()

        def _edge(e, c2):
            wsplat = plsc.load_gather(ew_v, [jnp.full((LANES,), e, jnp.int32)])

            def _cb(cb, c3):
                sl = pl.ds(cb * LANES, LANES)
                rows_v[e, sl] = rows_v[e, sl] * wsplat
                return c3

            lax.fori_loop(0, D // LANES, _cb, 0)
            return c2

        lax.fori_loop(0, C, _edge, 0)
        pltpu.sync_copy(rows_v, acc.at[dst_v], add=True)
        return carry

    lax.fori_loop(0, NCHUNK, _chunk, 0)
    plsc.subcore_barrier()

    def _flush(k, carry):
        r0 = sid * RPT + k * FR
        pltpu.sync_copy(acc.at[pl.ds(r0, FR)], zbuf)
        pltpu.sync_copy(zbuf, out_hbm.at[cid, pl.ds(r0, FR)])
        return carry

    lax.fori_loop(0, RPT // FR, _flush, 0)


_agg_call = pl.kernel(
    _agg_body,
    out_type=jax.ShapeDtypeStruct((NC, N, D), jnp.float32),
    mesh=_MESH,
    compiler_params=_SC_PARAMS,
    scratch_types=[
        pltpu.VMEM_SHARED((N, D), jnp.float32),
        pltpu.VMEM((C,), jnp.int32),
        pltpu.VMEM((C,), jnp.int32),
        pltpu.VMEM((C,), jnp.float32),
        pltpu.VMEM((C, D), jnp.float32),
        pltpu.VMEM((FR, D), jnp.float32),
        pltpu.SemaphoreType.DMA,
    ],
)


def _tc1_body(x_ref, w1_ref, degp_ref, g1_ref, dis_ref):
    deg = jnp.sum(degp_ref[...], axis=0) + 1.0
    dis = jnp.where(deg > 0, lax.rsqrt(deg), 0.0)
    h = jnp.dot(x_ref[...], w1_ref[...], preferred_element_type=jnp.float32)
    g1_ref[...] = h * dis[:, None]
    dis_ref[...] = dis


def _tc1(x, W1, degp):
    return pl.pallas_call(
        _tc1_body,
        out_shape=(
            jax.ShapeDtypeStruct((N, D), jnp.float32),
            jax.ShapeDtypeStruct((N,), jnp.float32),
        ),
    )(x, W1, degp)


def _tc2_body(p_ref, g1_ref, dis_ref, b1_ref, gnw_ref, gnb_ref, gnms_ref,
              w2_ref, g2_ref):
    dis = dis_ref[...]
    out1 = (p_ref[0] + p_ref[1] + g1_ref[...]) * dis[:, None] + b1_ref[...][None, :]
    mean = jnp.mean(out1, axis=0, keepdims=True)
    xc = out1 - gnms_ref[...][None, :] * mean
    var = jnp.mean(xc * xc, axis=0, keepdims=True)
    y = gnw_ref[...][None, :] * xc / jnp.sqrt(var + 1e-5) + gnb_ref[...][None, :]
    h2 = jnp.maximum(y, 0.0)
    hw = jnp.dot(h2, w2_ref[...], preferred_element_type=jnp.float32)
    g2_ref[...] = hw * dis[:, None]


def _tc2(p, g1, dis, b1, gn_weight, gn_bias, gn_mean_scale, W2):
    return pl.pallas_call(
        _tc2_body,
        out_shape=jax.ShapeDtypeStruct((N, D), jnp.float32),
    )(p, g1, dis, b1, gn_weight, gn_bias, gn_mean_scale, W2)


def _tc3_body(p_ref, g2_ref, dis_ref, b2_ref, out_ref):
    out_ref[...] = ((p_ref[0] + p_ref[1] + g2_ref[...])
                    * dis_ref[...][:, None] + b2_ref[...][None, :])


def _tc3(p, g2, dis, b2):
    return pl.pallas_call(
        _tc3_body,
        out_shape=jax.ShapeDtypeStruct((N, D), jnp.float32),
    )(p, g2, dis, b2)


def kernel(x, edge_index, edge_weight, W1, b1, gn_weight, gn_bias,
           gn_mean_scale, W2, b2):
    src = edge_index[0]
    dst = edge_index[1]
    degp = _deg_call(dst, edge_weight)
    g1, dis = _tc1(x, W1, degp)
    p1 = _agg_call(g1, src, dst, edge_weight)
    g2 = _tc2(p1, g1, dis, b1, gn_weight, gn_bias, gn_mean_scale, W2)
    p2 = _agg_call(g2, src, dst, edge_weight)
    out = _tc3(p2, g2, dis, b2)
    return out


# speedup vs baseline: 23.7214x; 2.6081x over previous
import jax
import jax.numpy as jnp
from jax import lax
from jax.experimental import pallas as pl
from jax.experimental.pallas import tpu as pltpu
from jax.experimental.pallas import tpu_sc as plsc

N = 10000
E = 320000
D = 128

NC = 2
NS = 16
NW = NC * NS
LANES = 16
EPT = E // NW
C = 80
NCHUNK = EPT // C
RPT = N // NS
FR = 125

_MESH = plsc.VectorSubcoreMesh(core_axis_name="c", subcore_axis_name="s")
_SC_PARAMS = pltpu.CompilerParams(needs_layout_passes=False,
                                  use_tc_tiling_on_sc=False)


def _worker_id():
    return lax.axis_index("s") * NC + lax.axis_index("c")


def _deg_body(dst_hbm, ew_hbm, out_hbm, deg_v, dst_a, ew_a):
    wid = _worker_id()
    pltpu.sync_copy(dst_hbm.at[wid], dst_a)
    pltpu.sync_copy(ew_hbm.at[wid], ew_a)
    zero16 = jnp.zeros((LANES,), jnp.float32)

    def _zero(i, carry):
        deg_v[pl.ds(i * LANES, LANES)] = zero16
        return carry

    lax.fori_loop(0, N // LANES, _zero, 0)

    def _grp(j, carry):
        idx = dst_a[pl.ds(j * LANES, LANES)]
        w = ew_a[pl.ds(j * LANES, LANES)]
        plsc.addupdate_scatter(deg_v, [idx], w)
        return carry

    lax.fori_loop(0, EPT // LANES, _grp, 0)
    pltpu.sync_copy(deg_v, out_hbm.at[wid])


_deg_call = pl.kernel(
    _deg_body,
    out_type=jax.ShapeDtypeStruct((NW, N), jnp.float32),
    mesh=_MESH,
    compiler_params=_SC_PARAMS,
    scratch_types=[
        pltpu.VMEM((N,), jnp.float32),
        pltpu.VMEM((EPT,), jnp.int32),
        pltpu.VMEM((EPT,), jnp.float32),
    ],
)


def _agg_body(g_hbm, src_hbm, dst_hbm, ew_hbm, out_hbm,
              acc, src_a, dst_a, ew_a, rows0, rows1, semg0, semg1):
    cid = lax.axis_index("c")
    sid = lax.axis_index("s")
    wid = sid * NC + cid
    zero16 = jnp.zeros((LANES,), jnp.float32)

    pltpu.sync_copy(src_hbm.at[wid], src_a)
    pltpu.sync_copy(dst_hbm.at[wid], dst_a)
    pltpu.sync_copy(ew_hbm.at[wid], ew_a)

    def _zr(r, carry):
        for cb in range(D // LANES):
            rows0[r, pl.ds(cb * LANES, LANES)] = zero16
        return carry

    lax.fori_loop(0, C, _zr, 0)

    def _za(k, carry):
        pltpu.sync_copy(rows0, acc.at[pl.ds(sid * RPT + k * C, C)])
        return carry

    lax.fori_loop(0, RPT // C, _za, 0)
    pltpu.sync_copy(rows0.at[pl.ds(0, RPT % C)],
                    acc.at[pl.ds(sid * RPT + (RPT // C) * C, RPT % C)])
    plsc.subcore_barrier()

    def _gather(i, buf, sem):
        return pltpu.make_async_copy(g_hbm.at[src_a.at[i]], buf, sem)

    def _process(i, buf, sem):
        _gather(i, buf, sem).wait()

        def _edge(e, carry):
            w = plsc.load_gather(
                ew_a, [jnp.full((LANES,), i, jnp.int32),
                       jnp.full((LANES,), e, jnp.int32)])
            for cb in range(D // LANES):
                sl = pl.ds(cb * LANES, LANES)
                buf[e, sl] = buf[e, sl] * w
            return carry

        lax.fori_loop(0, C, _edge, 0)
        pltpu.sync_copy(buf, acc.at[dst_a.at[i]], add=True)

        @pl.when(i + 2 < NCHUNK)
        def _():
            _gather(i + 2, buf, sem).start()

    _gather(0, rows0, semg0).start()
    _gather(1, rows1, semg1).start()

    def _pair(i2, carry):
        _process(2 * i2, rows0, semg0)
        _process(2 * i2 + 1, rows1, semg1)
        return carry

    lax.fori_loop(0, NCHUNK // 2, _pair, 0)
    _process(NCHUNK - 1, rows0, semg0)

    plsc.subcore_barrier()

    def _flush(k, carry):
        r0 = sid * RPT + k * C
        pltpu.sync_copy(acc.at[pl.ds(r0, C)], rows0)
        pltpu.sync_copy(rows0, out_hbm.at[cid, pl.ds(r0, C)])
        return carry

    lax.fori_loop(0, RPT // C, _flush, 0)
    rtail = RPT % C
    r0t = sid * RPT + (RPT // C) * C
    pltpu.sync_copy(acc.at[pl.ds(r0t, rtail)], rows0.at[pl.ds(0, rtail)])
    pltpu.sync_copy(rows0.at[pl.ds(0, rtail)], out_hbm.at[cid, pl.ds(r0t, rtail)])


_agg_call = pl.kernel(
    _agg_body,
    out_type=jax.ShapeDtypeStruct((NC, N, D), jnp.float32),
    mesh=_MESH,
    compiler_params=_SC_PARAMS,
    scratch_types=[
        pltpu.VMEM_SHARED((N, D), jnp.float32),
        pltpu.VMEM((NCHUNK, C), jnp.int32),
        pltpu.VMEM((NCHUNK, C), jnp.int32),
        pltpu.VMEM((NCHUNK, C), jnp.float32),
        pltpu.VMEM((C, D), jnp.float32),
        pltpu.VMEM((C, D), jnp.float32),
        pltpu.SemaphoreType.DMA,
        pltpu.SemaphoreType.DMA,
    ],
)


def _tc1_body(x_ref, w1_ref, degp_ref, g1_ref, dis_ref):
    deg = jnp.sum(degp_ref[...], axis=0) + 1.0
    dis = jnp.where(deg > 0, lax.rsqrt(deg), 0.0)
    h = jnp.dot(x_ref[...], w1_ref[...], preferred_element_type=jnp.float32)
    g1_ref[...] = h * dis[:, None]
    dis_ref[...] = dis


def _tc1(x, W1, degp):
    return pl.pallas_call(
        _tc1_body,
        out_shape=(
            jax.ShapeDtypeStruct((N, D), jnp.float32),
            jax.ShapeDtypeStruct((N,), jnp.float32),
        ),
    )(x, W1, degp)


def _tc2_body(p_ref, g1_ref, dis_ref, b1_ref, gnw_ref, gnb_ref, gnms_ref,
              w2_ref, g2_ref):
    dis = dis_ref[...]
    out1 = (p_ref[0] + p_ref[1] + g1_ref[...]) * dis[:, None] + b1_ref[...][None, :]
    mean = jnp.mean(out1, axis=0, keepdims=True)
    xc = out1 - gnms_ref[...][None, :] * mean
    var = jnp.mean(xc * xc, axis=0, keepdims=True)
    y = gnw_ref[...][None, :] * xc / jnp.sqrt(var + 1e-5) + gnb_ref[...][None, :]
    h2 = jnp.maximum(y, 0.0)
    hw = jnp.dot(h2, w2_ref[...], preferred_element_type=jnp.float32)
    g2_ref[...] = hw * dis[:, None]


def _tc2(p, g1, dis, b1, gn_weight, gn_bias, gn_mean_scale, W2):
    return pl.pallas_call(
        _tc2_body,
        out_shape=jax.ShapeDtypeStruct((N, D), jnp.float32),
    )(p, g1, dis, b1, gn_weight, gn_bias, gn_mean_scale, W2)


def _tc3_body(p_ref, g2_ref, dis_ref, b2_ref, out_ref):
    out_ref[...] = ((p_ref[0] + p_ref[1] + g2_ref[...])
                    * dis_ref[...][:, None] + b2_ref[...][None, :])


def _tc3(p, g2, dis, b2):
    return pl.pallas_call(
        _tc3_body,
        out_shape=jax.ShapeDtypeStruct((N, D), jnp.float32),
    )(p, g2, dis, b2)


def kernel(x, edge_index, edge_weight, W1, b1, gn_weight, gn_bias,
           gn_mean_scale, W2, b2):
    src = edge_index[0]
    dst = edge_index[1]
    src_c = src.reshape(NW, NCHUNK, C)
    dst_c = dst.reshape(NW, NCHUNK, C)
    ew_c = edge_weight.reshape(NW, NCHUNK, C)
    degp = _deg_call(dst.reshape(NW, EPT), edge_weight.reshape(NW, EPT))
    g1, dis = _tc1(x, W1, degp)
    p1 = _agg_call(g1, src_c, dst_c, ew_c)
    g2 = _tc2(p1, g1, dis, b1, gn_weight, gn_bias, gn_mean_scale, W2)
    p2 = _agg_call(g2, src_c, dst_c, ew_c)
    out = _tc3(p2, g2, dis, b2)
    return out
